# baseline (speedup 1.0000x reference)
import functools

import jax
import jax.numpy as jnp
from jax import lax
from jax.experimental import pallas as pl
from jax.experimental.pallas import tpu as pltpu, tpu_sc as plsc

NUM_CLASSES = 100000
EMB_DIM = 64
BATCH = 16384

_NW = 32
_NCOLS = 782
_TAILW = NUM_CLASSES - (_NCOLS - 1) * 128
_NGROUPS = BATCH // 16
_LISTCAP = BATCH + 32
_NSTAGE = 4


def _make_fused_gather():
    mesh = plsc.VectorSubcoreMesh(core_axis_name="c", subcore_axis_name="s")

    @functools.partial(
        pl.kernel,
        mesh=mesh,
        out_type=jax.ShapeDtypeStruct((BATCH + 16, 128), jnp.float32),
        scratch_types=[
            pltpu.VMEM((BATCH,), jnp.int32),
            pltpu.VMEM((_LISTCAP,), jnp.int32),
            pltpu.VMEM((_LISTCAP,), jnp.int32),
            pltpu.VMEM((2, EMB_DIM, 128), jnp.float32),
            pltpu.VMEM((_NSTAGE, 16, 128), jnp.float32),
            pltpu.VMEM((_NSTAGE, 16), jnp.int32),
            pltpu.SemaphoreType.DMA((2,)),
            pltpu.SemaphoreType.DMA((_NSTAGE,)),
            pltpu.SemaphoreType.DMA,
        ],
        compiler_params=pltpu.CompilerParams(needs_layout_passes=False),
    )
    def fused_kernel(idx_hbm, t2_hbm, tail_hbm, out_hbm, idx_v, mlist, cstage,
                     tiles_v, stage, sidx, sem_t, sem_o, sem_i):
        wid = lax.axis_index("s") * 2 + lax.axis_index("c")
        c_lo = wid * _NCOLS // _NW
        c_hi = (wid + 1) * _NCOLS // _NW
        full_hi = jnp.minimum(c_hi, _NCOLS - 1)
        iota = lax.iota(jnp.int32, 16)

        pltpu.make_async_copy(idx_hbm, idx_v, sem_i).start()

        def tile_copy(col, buf):
            s = pl.multiple_of(col * 128, 128)
            return pltpu.make_async_copy(
                t2_hbm.at[:, pl.ds(s, 128)], tiles_v.at[buf], sem_t.at[buf]
            )

        def tail_copy():
            return pltpu.make_async_copy(
                tail_hbm, tiles_v.at[0], sem_t.at[0]
            )

        tile_copy(c_lo, 0).start()
        pltpu.make_async_copy(idx_hbm, idx_v, sem_i).wait()

        def scan_body(g, n):
            v = idx_v[pl.ds(g * 16, 16)]
            cv = lax.shift_right_logical(v, 7)
            m = (cv >= c_lo) & (cv < c_hi)
            pk = v | lax.shift_left(iota + g * 16, 17)
            pref = plsc.cumsum(jnp.where(m, 1, 0))
            plsc.store_scatter(mlist, [n + pref - 1], pk, mask=m)
            return n + jnp.max(pref)

        n_match = lax.fori_loop(0, _NGROUPS, scan_body, 0)

        def process_block(col, buf, gcount):
            s = col * 128

            def rescan_body(g, cc):
                pk = mlist[pl.ds(g * 16, 16)]
                valid = (g * 16 + iota) < n_match
                cv = lax.shift_right_logical(pk & 0x1FFFF, 7)
                m = (cv == col) & valid
                pref = plsc.cumsum(jnp.where(m, 1, 0))
                plsc.store_scatter(cstage, [cc + pref - 1], pk, mask=m)
                return cc + jnp.max(pref)

            cnt = lax.fori_loop(0, (n_match + 15) >> 4, rescan_body, 0)

            def ex_body(g, gc):
                sb = lax.rem(gc, _NSTAGE)

                @pl.when(gc >= _NSTAGE)
                def _():
                    pltpu.make_async_copy(
                        stage.at[sb], out_hbm.at[sidx.at[sb]], sem_o.at[sb]
                    ).wait()

                pk = cstage[pl.ds(g * 16, 16)]
                cr = ((pk & 0x1FFFF) - s) & 127
                pos = lax.shift_right_logical(pk, 17)
                valid = (g * 16 + iota) < cnt
                pos_safe = jnp.where(valid, pos, BATCH)
                for rr in range(EMB_DIM):
                    rv = jnp.full((16,), rr, jnp.int32)
                    val = plsc.load_gather(tiles_v.at[buf], [rv, cr])
                    plsc.store_scatter(stage.at[sb], [iota, rv], val)
                sidx[sb, :] = pos_safe
                pltpu.make_async_copy(
                    stage.at[sb], out_hbm.at[sidx.at[sb]], sem_o.at[sb]
                ).start()
                return gc + 1

            return lax.fori_loop(0, (cnt + 15) >> 4, ex_body, gcount)

        def col_body(col, gcount):
            buf = lax.rem(col - c_lo, 2)
            tile_copy(col, buf).wait()

            @pl.when(col + 1 < full_hi)
            def _():
                tile_copy(col + 1, 1 - buf).start()

            return process_block(col, buf, gcount)

        gtotal = lax.fori_loop(c_lo, full_hi, col_body, 0)

        @pl.when(c_hi == _NCOLS)
        def _():
            tail_copy().start()
            tail_copy().wait()

        gtotal2 = lax.cond(
            c_hi == _NCOLS,
            lambda: process_block(_NCOLS - 1, 0, gtotal),
            lambda: gtotal,
        )

        def drain_body(k, _):
            sb = lax.rem(k, _NSTAGE)
            pltpu.make_async_copy(
                stage.at[sb], out_hbm.at[sidx.at[sb]], sem_o.at[sb]
            ).wait()
            return 0

        lax.fori_loop(jnp.maximum(gtotal2 - _NSTAGE, 0), gtotal2, drain_body, 0)

    return fused_kernel


_fused = _make_fused_gather()


@jax.jit
def kernel(class_indices, table):
    t2 = table.T
    tail = jnp.pad(t2[:, (_NCOLS - 1) * 128:], ((0, 0), (0, 128 - _TAILW)))
    padded = _fused(class_indices, t2, tail)
    return padded[:BATCH, :EMB_DIM]


# speedup vs baseline: 3.2345x; 3.2345x over previous
import functools

import jax
import jax.numpy as jnp
from jax import lax
from jax.experimental import pallas as pl
from jax.experimental.pallas import tpu as pltpu, tpu_sc as plsc

NUM_CLASSES = 100000
EMB_DIM = 64
BATCH = 16384

_NW = 32
_B_PER_W = BATCH // _NW
_CHUNK = 128
_NCHUNKS = _B_PER_W // _CHUNK


def _make_gather():
    mesh = plsc.VectorSubcoreMesh(core_axis_name="c", subcore_axis_name="s")

    @functools.partial(
        pl.kernel,
        mesh=mesh,
        out_type=jax.ShapeDtypeStruct((BATCH, 128), jnp.float32),
        scratch_types=[
            pltpu.VMEM((_NCHUNKS, _CHUNK), jnp.int32),
            pltpu.VMEM((_B_PER_W, 128), jnp.float32),
            pltpu.SemaphoreType.DMA,
        ],
    )
    def gather_kernel(idx_hbm, pairs_hbm, out_hbm, idx_v, rows_v, sem):
        wid = lax.axis_index("s") * 2 + lax.axis_index("c")
        base = wid * _B_PER_W
        pltpu.sync_copy(idx_hbm.at[wid], idx_v)
        for j in range(_NCHUNKS):
            for g in range(_CHUNK // 16):
                sl = pl.ds(g * 16, 16)
                idx_v[j, sl] = lax.shift_right_logical(idx_v[j, sl], 1)
        copies = []
        for j in range(_NCHUNKS):
            copies.append(
                pltpu.make_async_copy(
                    pairs_hbm.at[idx_v.at[j]],
                    rows_v.at[pl.ds(j * _CHUNK, _CHUNK)],
                    sem,
                )
            )
            copies[-1].start()
        for c in copies:
            c.wait()
        pltpu.sync_copy(rows_v, out_hbm.at[pl.ds(base, _B_PER_W)])

    return gather_kernel


_gather = _make_gather()


@jax.jit
def kernel(class_indices, table):
    pairs = table.reshape(NUM_CLASSES // 2, 2 * EMB_DIM)
    idx = class_indices.reshape(_NW, _NCHUNKS, _CHUNK)
    padded = _gather(idx, pairs)
    odd = (class_indices & 1).astype(jnp.bool_)
    return jnp.where(odd[:, None], padded[:, EMB_DIM:], padded[:, :EMB_DIM])


# speedup vs baseline: 4.0315x; 1.2464x over previous
import functools

import jax
import jax.numpy as jnp
from jax import lax
from jax.experimental import pallas as pl
from jax.experimental.pallas import tpu as pltpu, tpu_sc as plsc

NUM_CLASSES = 100000
EMB_DIM = 64
BATCH = 16384

_NW = 32
_B_PER_W = BATCH // _NW
_CHUNK = 128
_NCHUNKS = _B_PER_W // _CHUNK


def _make_gather():
    mesh = plsc.VectorSubcoreMesh(core_axis_name="c", subcore_axis_name="s")

    @functools.partial(
        pl.kernel,
        mesh=mesh,
        out_type=jax.ShapeDtypeStruct((BATCH, 128), jnp.float32),
        scratch_types=[
            pltpu.VMEM((_NCHUNKS, _CHUNK), jnp.int32),
            pltpu.VMEM((_B_PER_W, 128), jnp.float32),
            pltpu.SemaphoreType.DMA,
        ],
    )
    def gather_kernel(idx_hbm, tpad_hbm, out_hbm, idx_v, rows_v, sem):
        wid = lax.axis_index("s") * 2 + lax.axis_index("c")
        base = wid * _B_PER_W
        pltpu.sync_copy(idx_hbm.at[wid], idx_v)
        copies = []
        for j in range(_NCHUNKS):
            copies.append(
                pltpu.make_async_copy(
                    tpad_hbm.at[idx_v.at[j]],
                    rows_v.at[pl.ds(j * _CHUNK, _CHUNK)],
                    sem,
                )
            )
            copies[-1].start()
        for c in copies:
            c.wait()
        pltpu.sync_copy(rows_v, out_hbm.at[pl.ds(base, _B_PER_W)])

    return gather_kernel


_gather = _make_gather()


@jax.jit
def kernel(class_indices, table):
    tpad = jnp.pad(table, ((0, 0), (0, 128 - EMB_DIM)))
    idx = class_indices.reshape(_NW, _NCHUNKS, _CHUNK)
    padded = _gather(idx, tpad)
    return padded[:, :EMB_DIM]
